# baseline (speedup 1.0000x reference)
import functools

import jax
import jax.numpy as jnp
from jax import lax
from jax.experimental import pallas as pl
from jax.experimental.pallas import tpu as pltpu
from jax.experimental.pallas import tpu_sc as plsc

_INTERPRET = False
N = 10000
D = 128
NC = 2
NS = 16
NW = NC * NS
CHUNK = 128
K = 16
ROWS_PER_TILE = 624
LAST_EXTRA = N - ROWS_PER_TILE * NS
NCNT = 10240


def _zero_vmem_2d(ref, nrows, ncols):
    zs = jnp.zeros((16,), jnp.float32)
    npc = ncols // 16

    def body(t, _):
        for j in range(npc):
            ref[t, pl.ds(j * 16, 16)] = zs
        return 0

    lax.fori_loop(0, nrows, body, 0)


def _make_sc_aggregate(nchunk, ngroup, with_cnt):
    mesh = plsc.VectorSubcoreMesh(core_axis_name="c", subcore_axis_name="s")
    gbase = ngroup // NW
    gextra = ngroup - gbase * NW
    max_g = gbase + (1 if gextra else 0)

    out_type = [jax.ShapeDtypeStruct((NC, N, D), jnp.float32)]
    scratch = [
        pltpu.VMEM_SHARED((N + 8, D), jnp.float32),
        pltpu.VMEM((K, CHUNK), jnp.int32),
        pltpu.VMEM((K, CHUNK), jnp.int32),
        pltpu.VMEM((CHUNK, D), jnp.float32),
        pltpu.VMEM((CHUNK, D), jnp.float32),
        pltpu.SemaphoreType.DMA,
        pltpu.SemaphoreType.DMA,
        pltpu.SemaphoreType.DMA,
        pltpu.SemaphoreType.DMA,
    ]
    if with_cnt:
        out_type.append(jax.ShapeDtypeStruct((NW, NCNT), jnp.float32))
        scratch.append(pltpu.VMEM((NCNT,), jnp.float32))

    @functools.partial(
        pl.kernel, out_type=tuple(out_type), mesh=mesh, scratch_types=scratch,
        compiler_params=pltpu.CompilerParams(needs_layout_passes=False),
        interpret=_INTERPRET,
    )
    def sc_kernel(x_hbm, src_hbm, dst_hbm, *refs):
        if with_cnt:
            (agg_out, cnt_out, agg_sh, src_v, dst_v, rows_a, rows_b,
             gsem_a, gsem_b, ssem_a, ssem_b, cnt_v) = refs
        else:
            (agg_out, agg_sh, src_v, dst_v, rows_a, rows_b,
             gsem_a, gsem_b, ssem_a, ssem_b) = refs
        rows = (rows_a, rows_b)
        gsem = (gsem_a, gsem_b)
        ssem = (ssem_a, ssem_b)
        rows_v = rows_a

        c = lax.axis_index("c")
        s = lax.axis_index("s")
        wid = s * NC + c
        gstart = wid * gbase + jnp.minimum(wid, gextra)
        ng = gbase + jnp.where(wid < gextra, 1, 0)

        _zero_vmem_2d(rows_v, CHUNK, D)
        rbase = s * ROWS_PER_TILE
        for i in range(ROWS_PER_TILE // CHUNK):
            pltpu.sync_copy(rows_v, agg_sh.at[pl.ds(rbase + i * CHUNK, CHUNK)])
        rem = ROWS_PER_TILE % CHUNK
        if rem:
            pltpu.sync_copy(
                rows_v.at[pl.ds(0, rem)],
                agg_sh.at[pl.ds(rbase + ROWS_PER_TILE - rem, rem)],
            )

        @pl.when(s == NS - 1)
        def _():
            pltpu.sync_copy(
                rows_v.at[pl.ds(0, LAST_EXTRA)],
                agg_sh.at[pl.ds(ROWS_PER_TILE * NS, LAST_EXTRA)],
            )

        if with_cnt:
            zs = jnp.zeros((16,), jnp.float32)

            def zcnt(t, _):
                cnt_v[pl.ds(t * 16, 16)] = zs
                return 0

            lax.fori_loop(0, NCNT // 16, zcnt, 0)
        plsc.subcore_barrier()

        if with_cnt:
            lane = lax.iota(jnp.int32, 16)
            ones16 = jnp.ones((16,), jnp.float32)

        def group(g, _):
            @pl.when(g < ng)
            def _():
                gi = gstart + g
                pltpu.sync_copy(src_hbm.at[gi], src_v)
                pltpu.sync_copy(dst_hbm.at[gi], dst_v)
                gd = {}
                sd = {}
                for j in range(min(2, K)):
                    gd[j] = pltpu.async_copy(
                        x_hbm.at[src_v.at[j]], rows[j % 2], gsem[j % 2]
                    )
                for j in range(K):
                    b = j % 2
                    gd[j].wait()
                    sd[j] = pltpu.async_copy(
                        rows[b], agg_sh.at[dst_v.at[j]], ssem[b], add=True
                    )
                    if with_cnt:
                        def cnt_body(t, _, j=j):
                            idx16 = dst_v[j, pl.ds(t * 16, 16)]
                            for k in range(16):
                                plsc.addupdate_scatter(
                                    cnt_v, [idx16], ones16,
                                    mask=lane == k,
                                )
                            return 0

                        lax.fori_loop(0, CHUNK // 16, cnt_body, 0)
                    if j + 2 < K:
                        sd[j].wait()
                        gd[j + 2] = pltpu.async_copy(
                            x_hbm.at[src_v.at[j + 2]], rows[b], gsem[b]
                        )
                for j in range(max(0, K - 2), K):
                    sd[j].wait()
            return 0

        lax.fori_loop(0, max_g, group, 0)
        plsc.subcore_barrier()

        pltpu.sync_copy(
            agg_sh.at[pl.ds(rbase, ROWS_PER_TILE)],
            agg_out.at[c, pl.ds(rbase, ROWS_PER_TILE)],
        )

        @pl.when(s == NS - 1)
        def _():
            pltpu.sync_copy(
                agg_sh.at[pl.ds(ROWS_PER_TILE * NS, LAST_EXTRA)],
                agg_out.at[c, pl.ds(ROWS_PER_TILE * NS, LAST_EXTRA)],
            )

        if with_cnt:
            pltpu.sync_copy(cnt_v, cnt_out.at[wid])

    return sc_kernel


def _tc_layer_body(agg_ref, cnt_ref, x_ref, wlT_ref, bl_ref, wrT_ref, out_ref,
                   *, leaky, residual, br):
    agg = agg_ref[0] + agg_ref[1]
    cnt_row = jnp.sum(cnt_ref[...], axis=0, keepdims=True)
    iota_s = lax.broadcasted_iota(jnp.int32, (br, br), 0)
    iota_l = lax.broadcasted_iota(jnp.int32, (br, br), 1)
    cnt_b = jnp.broadcast_to(cnt_row, (br, br))
    cnt_col = jnp.sum(
        jnp.where(iota_s == iota_l, cnt_b, 0.0), axis=1, keepdims=True
    )
    mean = agg / jnp.maximum(cnt_col, 1.0)
    x = x_ref[...]
    out = (
        jnp.dot(mean, wlT_ref[...], preferred_element_type=jnp.float32)
        + bl_ref[...]
        + jnp.dot(x, wrT_ref[...], preferred_element_type=jnp.float32)
    )
    nrm = jnp.maximum(
        jnp.sqrt(jnp.sum(out * out, axis=-1, keepdims=True)), 1e-12
    )
    out = out / nrm
    if leaky:
        out = jnp.where(out >= 0, out, 0.01 * out)
    if residual:
        out = out + x
    out_ref[...] = out


def _tc_layer(agg, cnt, x, wlT, bl, wrT, *, leaky, residual, interpret=False):
    BR = 1024
    grid = ((N + BR - 1) // BR,)
    return pl.pallas_call(
        functools.partial(_tc_layer_body, leaky=leaky, residual=residual,
                          br=BR),
        grid=grid,
        in_specs=[
            pl.BlockSpec((NC, BR, D), lambda i: (0, i, 0)),
            pl.BlockSpec((NW, BR), lambda i: (0, i)),
            pl.BlockSpec((BR, D), lambda i: (i, 0)),
            pl.BlockSpec((D, D), lambda i: (0, 0)),
            pl.BlockSpec((1, D), lambda i: (0, 0)),
            pl.BlockSpec((D, D), lambda i: (0, 0)),
        ],
        out_specs=pl.BlockSpec((BR, D), lambda i: (i, 0)),
        out_shape=jax.ShapeDtypeStruct((N, D), jnp.float32),
        interpret=interpret,
    )(agg, cnt, x, wlT, bl, wrT)


@jax.jit
def kernel(x, edge_index, Wl1, bl1, Wr1, Wl2, bl2, Wr2):
    E = edge_index.shape[1]
    assert E % CHUNK == 0
    nchunk = E // CHUNK
    ngroup = (nchunk + K - 1) // K
    npad = ngroup * K
    src3d = jnp.zeros((npad, CHUNK), jnp.int32).at[:nchunk].set(
        edge_index[0].reshape(nchunk, CHUNK)).reshape(ngroup, K, CHUNK)
    dst3d = jnp.full((npad, CHUNK), N, jnp.int32).at[:nchunk].set(
        edge_index[1].reshape(nchunk, CHUNK)).reshape(ngroup, K, CHUNK)

    agg1, cnt = _make_sc_aggregate(nchunk, ngroup, True)(x, src3d, dst3d)
    h1 = _tc_layer(agg1, cnt, x, Wl1.T, bl1.reshape(1, D), Wr1.T,
                   leaky=True, residual=False)
    (agg2,) = _make_sc_aggregate(nchunk, ngroup, False)(h1, src3d, dst3d)
    out = _tc_layer(agg2, cnt, h1, Wl2.T, bl2.reshape(1, D), Wr2.T,
                    leaky=False, residual=True)
    return out


# speedup vs baseline: 1.1830x; 1.1830x over previous
import functools

import jax
import jax.numpy as jnp
from jax import lax
from jax.experimental import pallas as pl
from jax.experimental.pallas import tpu as pltpu
from jax.experimental.pallas import tpu_sc as plsc

_INTERPRET = False
N = 10000
D = 128
NC = 2
NS = 16
NW = NC * NS
CHUNK = 128
K = 8
ROWS_PER_TILE = 624
LAST_EXTRA = N - ROWS_PER_TILE * NS
NCNT = 10240


def _zero_vmem_2d(ref, nrows, ncols):
    zs = jnp.zeros((16,), jnp.float32)
    npc = ncols // 16

    def body(t, _):
        for j in range(npc):
            ref[t, pl.ds(j * 16, 16)] = zs
        return 0

    lax.fori_loop(0, nrows, body, 0)


def _make_sc_aggregate(nchunk, ngroup, with_cnt):
    mesh = plsc.VectorSubcoreMesh(core_axis_name="c", subcore_axis_name="s")
    gbase = ngroup // NW
    gextra = ngroup - gbase * NW
    max_g = gbase + (1 if gextra else 0)

    out_type = [jax.ShapeDtypeStruct((NC, N, D), jnp.float32)]
    scratch = [
        pltpu.VMEM_SHARED((N + 8, D), jnp.float32),
        pltpu.VMEM((K, CHUNK), jnp.int32),
        pltpu.VMEM((K, CHUNK), jnp.int32),
        pltpu.VMEM((CHUNK, D), jnp.float32),
        pltpu.VMEM((CHUNK, D), jnp.float32),
        pltpu.SemaphoreType.DMA,
        pltpu.SemaphoreType.DMA,
        pltpu.SemaphoreType.DMA,
        pltpu.SemaphoreType.DMA,
    ]
    if with_cnt:
        out_type.append(jax.ShapeDtypeStruct((NW, NCNT), jnp.float32))
        scratch.append(pltpu.VMEM((NCNT,), jnp.float32))

    @functools.partial(
        pl.kernel, out_type=tuple(out_type), mesh=mesh, scratch_types=scratch,
        compiler_params=pltpu.CompilerParams(needs_layout_passes=False),
        interpret=_INTERPRET,
    )
    def sc_kernel(x_hbm, src_hbm, dst_hbm, *refs):
        if with_cnt:
            (agg_out, cnt_out, agg_sh, src_v, dst_v, rows_a, rows_b,
             gsem_a, gsem_b, ssem_a, ssem_b, cnt_v) = refs
        else:
            (agg_out, agg_sh, src_v, dst_v, rows_a, rows_b,
             gsem_a, gsem_b, ssem_a, ssem_b) = refs
        rows = (rows_a, rows_b)
        gsem = (gsem_a, gsem_b)
        ssem = (ssem_a, ssem_b)
        rows_v = rows_a

        c = lax.axis_index("c")
        s = lax.axis_index("s")
        wid = s * NC + c
        gstart = wid * gbase + jnp.minimum(wid, gextra)
        ng = gbase + jnp.where(wid < gextra, 1, 0)

        _zero_vmem_2d(rows_v, CHUNK, D)
        rbase = s * ROWS_PER_TILE
        for i in range(ROWS_PER_TILE // CHUNK):
            pltpu.sync_copy(rows_v, agg_sh.at[pl.ds(rbase + i * CHUNK, CHUNK)])
        rem = ROWS_PER_TILE % CHUNK
        if rem:
            pltpu.sync_copy(
                rows_v.at[pl.ds(0, rem)],
                agg_sh.at[pl.ds(rbase + ROWS_PER_TILE - rem, rem)],
            )

        @pl.when(s == NS - 1)
        def _():
            pltpu.sync_copy(
                rows_v.at[pl.ds(0, LAST_EXTRA)],
                agg_sh.at[pl.ds(ROWS_PER_TILE * NS, LAST_EXTRA)],
            )

        if with_cnt:
            zs = jnp.zeros((16,), jnp.float32)

            def zcnt(t, _):
                cnt_v[pl.ds(t * 16, 16)] = zs
                return 0

            lax.fori_loop(0, NCNT // 16, zcnt, 0)
        plsc.subcore_barrier()

        if with_cnt:
            lane = lax.iota(jnp.int32, 16)
            ones16 = jnp.ones((16,), jnp.float32)

        def group(g, _):
            @pl.when(g < ng)
            def _():
                gi = gstart + g
                pltpu.sync_copy(src_hbm.at[gi], src_v)
                pltpu.sync_copy(dst_hbm.at[gi], dst_v)
                gd = {}
                sd = {}
                for j in range(min(2, K)):
                    gd[j] = pltpu.async_copy(
                        x_hbm.at[src_v.at[j]], rows[j % 2], gsem[j % 2]
                    )
                for j in range(K):
                    b = j % 2
                    gd[j].wait()
                    sd[j] = pltpu.async_copy(
                        rows[b], agg_sh.at[dst_v.at[j]], ssem[b], add=True
                    )
                    if with_cnt:
                        def cnt_body(t, _, j=j):
                            idx16 = dst_v[j, pl.ds(t * 16, 16)]
                            for k in range(16):
                                plsc.addupdate_scatter(
                                    cnt_v, [idx16], ones16,
                                    mask=lane == k,
                                )
                            return 0

                        lax.fori_loop(0, CHUNK // 16, cnt_body, 0)
                    if j + 2 < K:
                        sd[j].wait()
                        gd[j + 2] = pltpu.async_copy(
                            x_hbm.at[src_v.at[j + 2]], rows[b], gsem[b]
                        )
                for j in range(max(0, K - 2), K):
                    sd[j].wait()
            return 0

        lax.fori_loop(0, max_g, group, 0)
        plsc.subcore_barrier()

        pltpu.sync_copy(
            agg_sh.at[pl.ds(rbase, ROWS_PER_TILE)],
            agg_out.at[c, pl.ds(rbase, ROWS_PER_TILE)],
        )

        @pl.when(s == NS - 1)
        def _():
            pltpu.sync_copy(
                agg_sh.at[pl.ds(ROWS_PER_TILE * NS, LAST_EXTRA)],
                agg_out.at[c, pl.ds(ROWS_PER_TILE * NS, LAST_EXTRA)],
            )

        if with_cnt:
            pltpu.sync_copy(cnt_v, cnt_out.at[wid])

    return sc_kernel


def _tc_layer_body(agg_ref, cnt_ref, x_ref, wlT_ref, bl_ref, wrT_ref, out_ref,
                   *, leaky, residual, br):
    agg = agg_ref[0] + agg_ref[1]
    cnt_row = jnp.sum(cnt_ref[...], axis=0, keepdims=True)
    iota_s = lax.broadcasted_iota(jnp.int32, (br, br), 0)
    iota_l = lax.broadcasted_iota(jnp.int32, (br, br), 1)
    cnt_b = jnp.broadcast_to(cnt_row, (br, br))
    cnt_col = jnp.sum(
        jnp.where(iota_s == iota_l, cnt_b, 0.0), axis=1, keepdims=True
    )
    mean = agg / jnp.maximum(cnt_col, 1.0)
    x = x_ref[...]
    out = (
        jnp.dot(mean, wlT_ref[...], preferred_element_type=jnp.float32)
        + bl_ref[...]
        + jnp.dot(x, wrT_ref[...], preferred_element_type=jnp.float32)
    )
    nrm = jnp.maximum(
        jnp.sqrt(jnp.sum(out * out, axis=-1, keepdims=True)), 1e-12
    )
    out = out / nrm
    if leaky:
        out = jnp.where(out >= 0, out, 0.01 * out)
    if residual:
        out = out + x
    out_ref[...] = out


def _tc_layer(agg, cnt, x, wlT, bl, wrT, *, leaky, residual, interpret=False):
    BR = 1024
    grid = ((N + BR - 1) // BR,)
    return pl.pallas_call(
        functools.partial(_tc_layer_body, leaky=leaky, residual=residual,
                          br=BR),
        grid=grid,
        in_specs=[
            pl.BlockSpec((NC, BR, D), lambda i: (0, i, 0)),
            pl.BlockSpec((NW, BR), lambda i: (0, i)),
            pl.BlockSpec((BR, D), lambda i: (i, 0)),
            pl.BlockSpec((D, D), lambda i: (0, 0)),
            pl.BlockSpec((1, D), lambda i: (0, 0)),
            pl.BlockSpec((D, D), lambda i: (0, 0)),
        ],
        out_specs=pl.BlockSpec((BR, D), lambda i: (i, 0)),
        out_shape=jax.ShapeDtypeStruct((N, D), jnp.float32),
        interpret=interpret,
    )(agg, cnt, x, wlT, bl, wrT)


@jax.jit
def kernel(x, edge_index, Wl1, bl1, Wr1, Wl2, bl2, Wr2):
    E = edge_index.shape[1]
    assert E % CHUNK == 0
    nchunk = E // CHUNK
    ngroup = (nchunk + K - 1) // K
    npad = ngroup * K
    src3d = jnp.zeros((npad, CHUNK), jnp.int32).at[:nchunk].set(
        edge_index[0].reshape(nchunk, CHUNK)).reshape(ngroup, K, CHUNK)
    dst3d = jnp.full((npad, CHUNK), N, jnp.int32).at[:nchunk].set(
        edge_index[1].reshape(nchunk, CHUNK)).reshape(ngroup, K, CHUNK)

    agg1, cnt = _make_sc_aggregate(nchunk, ngroup, True)(x, src3d, dst3d)
    h1 = _tc_layer(agg1, cnt, x, Wl1.T, bl1.reshape(1, D), Wr1.T,
                   leaky=True, residual=False)
    (agg2,) = _make_sc_aggregate(nchunk, ngroup, False)(h1, src3d, dst3d)
    out = _tc_layer(agg2, cnt, h1, Wl2.T, bl2.reshape(1, D), Wr2.T,
                    leaky=False, residual=True)
    return out


# speedup vs baseline: 1.1846x; 1.0014x over previous
import functools

import jax
import jax.numpy as jnp
from jax import lax
from jax.experimental import pallas as pl
from jax.experimental.pallas import tpu as pltpu
from jax.experimental.pallas import tpu_sc as plsc

N = 10000
D = 128
NC = 2
NS = 16
NW = NC * NS
CHUNK = 128
K = 8
ROWS_PER_TILE = 624
LAST_EXTRA = N - ROWS_PER_TILE * NS
NCNT = 10240


def _zero_vmem_2d(ref, nrows, ncols):
    zs = jnp.zeros((16,), jnp.float32)
    npc = ncols // 16

    def body(t, _):
        for j in range(npc):
            ref[t, pl.ds(j * 16, 16)] = zs
        return 0

    lax.fori_loop(0, nrows, body, 0)


def _make_sc_aggregate(nchunk, ngroup, with_cnt):
    mesh = plsc.VectorSubcoreMesh(core_axis_name="c", subcore_axis_name="s")
    gbase = ngroup // NW
    gextra = ngroup - gbase * NW
    max_g = gbase + (1 if gextra else 0)

    out_type = [jax.ShapeDtypeStruct((NC, N, D), jnp.float32)]
    scratch = [
        pltpu.VMEM_SHARED((N + 8, D), jnp.float32),
        pltpu.VMEM((K, CHUNK), jnp.int32),
        pltpu.VMEM((K, CHUNK), jnp.int32),
        pltpu.VMEM((CHUNK, D), jnp.float32),
        pltpu.VMEM((CHUNK, D), jnp.float32),
        pltpu.SemaphoreType.DMA,
        pltpu.SemaphoreType.DMA,
        pltpu.SemaphoreType.DMA,
        pltpu.SemaphoreType.DMA,
    ]
    if with_cnt:
        out_type.append(jax.ShapeDtypeStruct((NW, NCNT), jnp.float32))
        scratch.append(pltpu.VMEM((NCNT,), jnp.float32))

    @functools.partial(
        pl.kernel, out_type=tuple(out_type), mesh=mesh, scratch_types=scratch,
        compiler_params=pltpu.CompilerParams(needs_layout_passes=False),
    )
    def sc_kernel(x_hbm, src_hbm, dst_hbm, *refs):
        if with_cnt:
            (agg_out, cnt_out, agg_sh, src_v, dst_v, rows_a, rows_b,
             gsem_a, gsem_b, ssem_a, ssem_b, cnt_v) = refs
        else:
            (agg_out, agg_sh, src_v, dst_v, rows_a, rows_b,
             gsem_a, gsem_b, ssem_a, ssem_b) = refs
        rows = (rows_a, rows_b)
        gsem = (gsem_a, gsem_b)
        ssem = (ssem_a, ssem_b)
        rows_v = rows_a

        c = lax.axis_index("c")
        s = lax.axis_index("s")
        wid = s * NC + c
        gstart = wid * gbase + jnp.minimum(wid, gextra)
        ng = gbase + jnp.where(wid < gextra, 1, 0)

        _zero_vmem_2d(rows_v, CHUNK, D)
        rbase = s * ROWS_PER_TILE
        for i in range(ROWS_PER_TILE // CHUNK):
            pltpu.sync_copy(rows_v, agg_sh.at[pl.ds(rbase + i * CHUNK, CHUNK)])
        rem = ROWS_PER_TILE % CHUNK
        if rem:
            pltpu.sync_copy(
                rows_v.at[pl.ds(0, rem)],
                agg_sh.at[pl.ds(rbase + ROWS_PER_TILE - rem, rem)],
            )

        @pl.when(s == NS - 1)
        def _():
            pltpu.sync_copy(
                rows_v.at[pl.ds(0, LAST_EXTRA)],
                agg_sh.at[pl.ds(ROWS_PER_TILE * NS, LAST_EXTRA)],
            )

        if with_cnt:
            zs = jnp.zeros((16,), jnp.float32)

            def zcnt(t, _):
                cnt_v[pl.ds(t * 16, 16)] = zs
                return 0

            lax.fori_loop(0, NCNT // 16, zcnt, 0)
        plsc.subcore_barrier()

        if with_cnt:
            ones16 = jnp.ones((16,), jnp.float32)

        def group(g, _):
            @pl.when(g < ng)
            def _():
                gi = gstart + g
                pltpu.sync_copy(src_hbm.at[gi], src_v)
                pltpu.sync_copy(dst_hbm.at[gi], dst_v)
                gd = {}
                sd = {}
                for j in range(min(2, K)):
                    gd[j] = pltpu.async_copy(
                        x_hbm.at[src_v.at[j]], rows[j % 2], gsem[j % 2]
                    )
                for j in range(K):
                    b = j % 2
                    gd[j].wait()
                    sd[j] = pltpu.async_copy(
                        rows[b], agg_sh.at[dst_v.at[j]], ssem[b], add=True
                    )
                    if with_cnt:
                        def cnt_body(t, _, j=j):
                            idx16 = dst_v[j, pl.ds(t * 16, 16)]
                            plsc.addupdate_scatter(cnt_v, [idx16], ones16)
                            return 0

                        lax.fori_loop(0, CHUNK // 16, cnt_body, 0)
                    if j + 2 < K:
                        sd[j].wait()
                        gd[j + 2] = pltpu.async_copy(
                            x_hbm.at[src_v.at[j + 2]], rows[b], gsem[b]
                        )
                for j in range(max(0, K - 2), K):
                    sd[j].wait()
            return 0

        lax.fori_loop(0, max_g, group, 0)
        plsc.subcore_barrier()

        pltpu.sync_copy(
            agg_sh.at[pl.ds(rbase, ROWS_PER_TILE)],
            agg_out.at[c, pl.ds(rbase, ROWS_PER_TILE)],
        )

        @pl.when(s == NS - 1)
        def _():
            pltpu.sync_copy(
                agg_sh.at[pl.ds(ROWS_PER_TILE * NS, LAST_EXTRA)],
                agg_out.at[c, pl.ds(ROWS_PER_TILE * NS, LAST_EXTRA)],
            )

        if with_cnt:
            pltpu.sync_copy(cnt_v, cnt_out.at[wid])

    return sc_kernel


def _tc_layer_body(agg_ref, cnt_ref, x_ref, wlT_ref, bl_ref, wrT_ref, out_ref,
                   *, leaky, residual, br):
    agg = agg_ref[0] + agg_ref[1]
    cnt_row = jnp.sum(cnt_ref[...], axis=0, keepdims=True)
    iota_s = lax.broadcasted_iota(jnp.int32, (br, br), 0)
    iota_l = lax.broadcasted_iota(jnp.int32, (br, br), 1)
    cnt_b = jnp.broadcast_to(cnt_row, (br, br))
    cnt_col = jnp.sum(
        jnp.where(iota_s == iota_l, cnt_b, 0.0), axis=1, keepdims=True
    )
    mean = agg / jnp.maximum(cnt_col, 1.0)
    x = x_ref[...]
    out = (
        jnp.dot(mean, wlT_ref[...], preferred_element_type=jnp.float32)
        + bl_ref[...]
        + jnp.dot(x, wrT_ref[...], preferred_element_type=jnp.float32)
    )
    nrm = jnp.maximum(
        jnp.sqrt(jnp.sum(out * out, axis=-1, keepdims=True)), 1e-12
    )
    out = out / nrm
    if leaky:
        out = jnp.where(out >= 0, out, 0.01 * out)
    if residual:
        out = out + x
    out_ref[...] = out


def _tc_layer(agg, cnt, x, wlT, bl, wrT, *, leaky, residual, interpret=False):
    BR = 1024
    grid = ((N + BR - 1) // BR,)
    return pl.pallas_call(
        functools.partial(_tc_layer_body, leaky=leaky, residual=residual,
                          br=BR),
        grid=grid,
        in_specs=[
            pl.BlockSpec((NC, BR, D), lambda i: (0, i, 0)),
            pl.BlockSpec((NW, BR), lambda i: (0, i)),
            pl.BlockSpec((BR, D), lambda i: (i, 0)),
            pl.BlockSpec((D, D), lambda i: (0, 0)),
            pl.BlockSpec((1, D), lambda i: (0, 0)),
            pl.BlockSpec((D, D), lambda i: (0, 0)),
        ],
        out_specs=pl.BlockSpec((BR, D), lambda i: (i, 0)),
        out_shape=jax.ShapeDtypeStruct((N, D), jnp.float32),
        interpret=interpret,
    )(agg, cnt, x, wlT, bl, wrT)


@jax.jit
def kernel(x, edge_index, Wl1, bl1, Wr1, Wl2, bl2, Wr2):
    E = edge_index.shape[1]
    assert E % CHUNK == 0
    nchunk = E // CHUNK
    ngroup = (nchunk + K - 1) // K
    npad = ngroup * K
    src3d = jnp.zeros((npad, CHUNK), jnp.int32).at[:nchunk].set(
        edge_index[0].reshape(nchunk, CHUNK)).reshape(ngroup, K, CHUNK)
    dst3d = jnp.full((npad, CHUNK), N, jnp.int32).at[:nchunk].set(
        edge_index[1].reshape(nchunk, CHUNK)).reshape(ngroup, K, CHUNK)

    agg1, cnt = _make_sc_aggregate(nchunk, ngroup, True)(x, src3d, dst3d)
    h1 = _tc_layer(agg1, cnt, x, Wl1.T, bl1.reshape(1, D), Wr1.T,
                   leaky=True, residual=False)
    (agg2,) = _make_sc_aggregate(nchunk, ngroup, False)(h1, src3d, dst3d)
    out = _tc_layer(agg2, cnt, h1, Wl2.T, bl2.reshape(1, D), Wr2.T,
                    leaky=False, residual=True)
    return out
